# baseline (speedup 1.0000x reference)
import functools

import jax
import jax.numpy as jnp
from jax import lax
from jax.experimental import pallas as pl
from jax.experimental.pallas import tpu as pltpu
from jax.experimental.pallas import tpu_sc as plsc

_LANES = 16


@functools.cache
def _build(B, S, N, D):
    info = plsc.get_sparse_core_info()
    nw = info.num_cores * info.num_subcores
    rows = S // nw

    mesh = plsc.VectorSubcoreMesh(core_axis_name="c", subcore_axis_name="s")

    rpg = _LANES // N
    xvecs = rows // rpg

    @functools.partial(
        pl.kernel,
        mesh=mesh,
        out_type=jax.ShapeDtypeStruct((B, S, D), jnp.float32),
        scratch_types=[
            pltpu.VMEM((N, D), jnp.float32),
            pltpu.VMEM((rows, D), jnp.float32),
            pltpu.VMEM((xvecs, _LANES), jnp.float32),
            pltpu.VMEM((rows, D), jnp.float32),
        ],
    )
    def sc_kernel(x_hbm, a_hbm, p_hbm, out_hbm, a_v, p_v, x_v, o_v):
        cid = lax.axis_index("c")
        sid = lax.axis_index("s")
        wid = sid * info.num_cores + cid
        s0 = wid * rows

        pltpu.sync_copy(a_hbm, a_v)
        pltpu.sync_copy(p_hbm.at[pl.ds(s0, rows)], p_v)

        nj = D // _LANES
        a_regs = [[a_v[n, pl.ds(j * _LANES, _LANES)] for j in range(nj)]
                  for n in range(N)]

        unroll = 2

        def batch_body(b, carry):
            pltpu.sync_copy(x_hbm.at[b, pl.ds(wid * xvecs, xvecs)], x_v)

            def grp_body(g, carry2):
                for u in range(unroll):
                    q = g * unroll + u
                    xv = x_v[q, :]
                    for t in range(rpg):
                        r = q * rpg + t
                        for j in range(nj):
                            sl = pl.ds(j * _LANES, _LANES)
                            acc = p_v[r, sl]
                            for n in range(N):
                                acc = acc + xv[t * N + n] * a_regs[n][j]
                            o_v[r, sl] = acc
                return carry2

            lax.fori_loop(0, xvecs // unroll, grp_body, 0)
            pltpu.sync_copy(o_v, out_hbm.at[b, pl.ds(s0, rows)])
            return carry

        lax.fori_loop(0, B, batch_body, 0)

    return sc_kernel


def kernel(x, allele_embedding, position_table):
    B, S, N = x.shape
    D = allele_embedding.shape[1]
    x2 = x.reshape(B, (S * N) // _LANES, _LANES)
    return _build(B, S, N, D)(x2, allele_embedding, position_table)


# speedup vs baseline: 1.2944x; 1.2944x over previous
import functools

import jax
import jax.numpy as jnp
from jax import lax
from jax.experimental import pallas as pl
from jax.experimental.pallas import tpu as pltpu
from jax.experimental.pallas import tpu_sc as plsc

_LANES = 16


@functools.cache
def _build(B, S, N, D):
    info = plsc.get_sparse_core_info()
    nw = info.num_cores * info.num_subcores
    rows = S // nw

    mesh = plsc.VectorSubcoreMesh(core_axis_name="c", subcore_axis_name="s")

    rpg = _LANES // N
    xvecs = rows // rpg
    nj = D // _LANES

    @functools.partial(
        pl.kernel,
        mesh=mesh,
        out_type=jax.ShapeDtypeStruct((B, S, D), jnp.float32),
        scratch_types=[
            pltpu.VMEM((N, D), jnp.float32),
            pltpu.VMEM((rows, D), jnp.float32),
            pltpu.VMEM((xvecs, _LANES), jnp.float32),
            pltpu.VMEM((xvecs, _LANES), jnp.float32),
            pltpu.VMEM((rows, D), jnp.float32),
            pltpu.VMEM((rows, D), jnp.float32),
            pltpu.SemaphoreType.DMA,
            pltpu.SemaphoreType.DMA,
            pltpu.SemaphoreType.DMA,
            pltpu.SemaphoreType.DMA,
        ],
    )
    def sc_kernel(x_hbm, a_hbm, p_hbm, out_hbm,
                  a_v, p_v, x0_v, x1_v, o0_v, o1_v,
                  sx0, sx1, so0, so1):
        cid = lax.axis_index("c")
        sid = lax.axis_index("s")
        wid = sid * info.num_cores + cid
        s0 = wid * rows
        xs0 = wid * xvecs

        pltpu.sync_copy(a_hbm, a_v)
        pltpu.sync_copy(p_hbm.at[pl.ds(s0, rows)], p_v)

        a_regs = [[a_v[n, pl.ds(j * _LANES, _LANES)] for j in range(nj)]
                  for n in range(N)]

        unroll = 2

        def compute(x_v, o_v):
            def grp_body(g, carry2):
                for u in range(unroll):
                    q = g * unroll + u
                    xv = x_v[q, :]
                    for t in range(rpg):
                        r = q * rpg + t
                        for j in range(nj):
                            sl = pl.ds(j * _LANES, _LANES)
                            acc = p_v[r, sl]
                            for n in range(N):
                                acc = acc + xv[t * N + n] * a_regs[n][j]
                            o_v[r, sl] = acc
                return carry2

            lax.fori_loop(0, xvecs // unroll, grp_body, 0)

        def fetch_x(b, x_v, sem):
            bc = jnp.minimum(b, B - 1)
            pltpu.async_copy(x_hbm.at[bc, pl.ds(xs0, xvecs)], x_v, sem)

        def wait_x(x_v, sem):
            pltpu.make_async_copy(x_hbm.at[0, pl.ds(xs0, xvecs)], x_v, sem).wait()

        def wait_out(o_v, sem):
            pltpu.make_async_copy(o_v, out_hbm.at[0, pl.ds(s0, rows)], sem).wait()

        fetch_x(0, x0_v, sx0)

        def batch_pair(g, carry):
            b0 = 2 * g
            fetch_x(b0 + 1, x1_v, sx1)
            wait_x(x0_v, sx0)

            @pl.when(g > 0)
            def _():
                wait_out(o0_v, so0)

            compute(x0_v, o0_v)
            pltpu.async_copy(o0_v, out_hbm.at[b0, pl.ds(s0, rows)], so0)

            fetch_x(b0 + 2, x0_v, sx0)
            wait_x(x1_v, sx1)

            @pl.when(g > 0)
            def _():
                wait_out(o1_v, so1)

            compute(x1_v, o1_v)
            pltpu.async_copy(o1_v, out_hbm.at[b0 + 1, pl.ds(s0, rows)], so1)
            return carry

        lax.fori_loop(0, B // 2, batch_pair, 0)

        wait_x(x0_v, sx0)
        wait_out(o0_v, so0)
        wait_out(o1_v, so1)

    return sc_kernel


def kernel(x, allele_embedding, position_table):
    B, S, N = x.shape
    D = allele_embedding.shape[1]
    x2 = x.reshape(B, (S * N) // _LANES, _LANES)
    return _build(B, S, N, D)(x2, allele_embedding, position_table)


# speedup vs baseline: 1.2945x; 1.0000x over previous
import functools

import jax
import jax.numpy as jnp
from jax import lax
from jax.experimental import pallas as pl
from jax.experimental.pallas import tpu as pltpu
from jax.experimental.pallas import tpu_sc as plsc

_LANES = 16


@functools.cache
def _build(B, S, N, D):
    info = plsc.get_sparse_core_info()
    nw = info.num_cores * info.num_subcores
    rows = S // nw

    mesh = plsc.VectorSubcoreMesh(core_axis_name="c", subcore_axis_name="s")

    rpg = _LANES // N
    xvecs = rows // rpg
    nj = D // _LANES

    @functools.partial(
        pl.kernel,
        mesh=mesh,
        out_type=jax.ShapeDtypeStruct((B, S, D), jnp.float32),
        scratch_types=[
            pltpu.VMEM((N, D), jnp.float32),
            pltpu.VMEM((rows, D), jnp.float32),
            pltpu.VMEM((xvecs, _LANES), jnp.float32),
            pltpu.VMEM((xvecs, _LANES), jnp.float32),
            pltpu.VMEM((rows, D), jnp.float32),
            pltpu.VMEM((rows, D), jnp.float32),
            pltpu.SemaphoreType.DMA,
            pltpu.SemaphoreType.DMA,
            pltpu.SemaphoreType.DMA,
            pltpu.SemaphoreType.DMA,
        ],
    )
    def sc_kernel(x_hbm, a_hbm, p_hbm, out_hbm,
                  a_v, p_v, x0_v, x1_v, o0_v, o1_v,
                  sx0, sx1, so0, so1):
        cid = lax.axis_index("c")
        sid = lax.axis_index("s")
        wid = sid * info.num_cores + cid
        s0 = wid * rows
        xs0 = wid * xvecs

        pltpu.sync_copy(a_hbm, a_v)
        pltpu.sync_copy(p_hbm.at[pl.ds(s0, rows)], p_v)

        a_regs = [[a_v[n, pl.ds(j * _LANES, _LANES)] for j in range(nj)]
                  for n in range(N)]

        unroll = 2

        bidx = [jnp.full((_LANES, 1), k, jnp.int32) for k in range(_LANES)]
        gdn = lax.GatherDimensionNumbers(
            offset_dims=(), collapsed_slice_dims=(0,), start_index_map=(0,))

        def bcast(xv, k):
            return lax.gather(xv, bidx[k], gdn, (1,),
                              mode=lax.GatherScatterMode.PROMISE_IN_BOUNDS)

        def compute(x_v, o_v):
            def grp_body(g, carry2):
                for u in range(unroll):
                    q = g * unroll + u
                    xv = x_v[q, :]
                    xb = [bcast(xv, k) for k in range(_LANES)]
                    for t in range(rpg):
                        r = q * rpg + t
                        for j in range(nj):
                            sl = pl.ds(j * _LANES, _LANES)
                            acc = p_v[r, sl]
                            for n in range(N):
                                acc = acc + xb[t * N + n] * a_regs[n][j]
                            o_v[r, sl] = acc
                return carry2

            lax.fori_loop(0, xvecs // unroll, grp_body, 0)

        def fetch_x(b, x_v, sem):
            bc = jnp.minimum(b, B - 1)
            pltpu.async_copy(x_hbm.at[bc, pl.ds(xs0, xvecs)], x_v, sem)

        def wait_x(x_v, sem):
            pltpu.make_async_copy(x_hbm.at[0, pl.ds(xs0, xvecs)], x_v, sem).wait()

        def wait_out(o_v, sem):
            pltpu.make_async_copy(o_v, out_hbm.at[0, pl.ds(s0, rows)], sem).wait()

        fetch_x(0, x0_v, sx0)

        def batch_pair(g, carry):
            b0 = 2 * g
            fetch_x(b0 + 1, x1_v, sx1)
            wait_x(x0_v, sx0)

            @pl.when(g > 0)
            def _():
                wait_out(o0_v, so0)

            compute(x0_v, o0_v)
            pltpu.async_copy(o0_v, out_hbm.at[b0, pl.ds(s0, rows)], so0)

            fetch_x(b0 + 2, x0_v, sx0)
            wait_x(x1_v, sx1)

            @pl.when(g > 0)
            def _():
                wait_out(o1_v, so1)

            compute(x1_v, o1_v)
            pltpu.async_copy(o1_v, out_hbm.at[b0 + 1, pl.ds(s0, rows)], so1)
            return carry

        lax.fori_loop(0, B // 2, batch_pair, 0)

        wait_x(x0_v, sx0)
        wait_out(o0_v, so0)
        wait_out(o1_v, so1)

    return sc_kernel


def kernel(x, allele_embedding, position_table):
    B, S, N = x.shape
    D = allele_embedding.shape[1]
    x2 = x.reshape(B, (S * N) // _LANES, _LANES)
    return _build(B, S, N, D)(x2, allele_embedding, position_table)
